# Initial kernel scaffold; baseline (speedup 1.0000x reference)
import jax
import jax.numpy as jnp
from jax.experimental import pallas as pl


def kernel(x, edge_index, edge_attr, batch, params):
    raise NotImplementedError("write your pallas kernel here")


# speedup vs baseline: 1.1816x; 1.1816x over previous
import functools

import jax
import jax.numpy as jnp
from jax import lax
from jax.experimental import pallas as pl
from jax.experimental.pallas import tpu as pltpu
from jax.experimental.pallas import tpu_sc as plsc

L = 16
NCORES = 2
NSUB = 16
NW = NCORES * NSUB
CHUNK = 4000


def _take(x, idx):
    dn = lax.GatherDimensionNumbers(
        offset_dims=(), collapsed_slice_dims=(0,), start_index_map=(0,))
    return lax.gather(x, idx[:, None], dn, (1,),
                      mode=lax.GatherScatterMode.PROMISE_IN_BOUNDS)


def _make_segmax(H, NP, E, use_ew):
    assert H % NW == 0
    fpt = H // NW
    assert E % (2 * CHUNK) == 0
    nchunk = E // CHUNK
    nsteps = CHUNK // L

    mesh = plsc.VectorSubcoreMesh(core_axis_name="c", subcore_axis_name="s",
                                  num_cores=NCORES, num_subcores=NSUB)

    scratch = [
        pltpu.VMEM((fpt, NP), jnp.float32),
        pltpu.VMEM((fpt, NP), jnp.float32),
        pltpu.VMEM((CHUNK,), jnp.int32),
        pltpu.VMEM((CHUNK,), jnp.int32),
        pltpu.VMEM((CHUNK,), jnp.int32),
        pltpu.VMEM((CHUNK,), jnp.int32),
        pltpu.SemaphoreType.DMA,
        pltpu.SemaphoreType.DMA,
    ]
    if use_ew:
        scratch += [pltpu.VMEM((CHUNK,), jnp.float32),
                    pltpu.VMEM((CHUNK,), jnp.float32)]

    def body(hT, src, dst, *rest):
        if use_ew:
            (ew, out, h_loc, agg, src0, src1, dst0, dst1, sem0, sem1,
             ew0, ew1) = rest
            ewb = (ew0, ew1)
        else:
            out, h_loc, agg, src0, src1, dst0, dst1, sem0, sem1 = rest
        srcb = (src0, src1)
        dstb = (dst0, dst1)
        sems = (sem0, sem1)

        wid = lax.axis_index("s") * NCORES + lax.axis_index("c")
        f0 = wid * fpt

        def fire(slot, c):
            base = c * CHUNK
            pltpu.async_copy(src.at[pl.ds(base, CHUNK)], srcb[slot], sems[slot])
            pltpu.async_copy(dst.at[pl.ds(base, CHUNK)], dstb[slot], sems[slot])
            if use_ew:
                pltpu.async_copy(ew.at[pl.ds(base, CHUNK)], ewb[slot], sems[slot])

        def drain(slot):
            pltpu.make_async_copy(src.at[pl.ds(0, CHUNK)], srcb[slot], sems[slot]).wait()
            pltpu.make_async_copy(dst.at[pl.ds(0, CHUNK)], dstb[slot], sems[slot]).wait()
            if use_ew:
                pltpu.make_async_copy(ew.at[pl.ds(0, CHUNK)], ewb[slot], sems[slot]).wait()

        fire(0, 0)
        fire(1, 1)

        pltpu.sync_copy(hT.at[pl.ds(f0, fpt)], h_loc)

        neg_inf = jnp.float32(-jnp.inf)
        filler = jnp.full((L,), neg_inf, jnp.float32)

        @pl.loop(0, NP // L)
        def _(i):
            for f in range(fpt):
                agg[f, pl.ds(i * L, L)] = filler

        lane = lax.iota(jnp.int32, L)
        idx_up = jnp.maximum(lane - 1, 0)
        idx_dn = jnp.minimum(lane + 1, L - 1)
        lanepos = lane > 0
        lanelast = lane == (L - 1)
        log_steps = [(jnp.maximum(lane - s, 0), lane >= s) for s in (1, 2, 4, 8)]
        fsplat = [jnp.full((L,), f, jnp.int32) for f in range(fpt)]

        @pl.loop(0, nchunk, step=2)
        def _(c0):
            for slot in range(2):
                c = c0 + slot
                drain(slot)

                @pl.loop(0, nsteps)
                def _(i):
                    b = i * L
                    s16 = srcb[slot][pl.ds(b, L)]
                    d16 = dstb[slot][pl.ds(b, L)]
                    w16 = ewb[slot][pl.ds(b, L)] if use_ew else None
                    d_s, perm = plsc.sort_key_val(d16, lane)
                    dup = (d_s == _take(d_s, idx_up)) & lanepos
                    ndup = jnp.max(jnp.where(dup, 1, 0))

                    @pl.when(ndup == 0)
                    def _fast():
                        for f in range(fpt):
                            v = plsc.load_gather(h_loc, [fsplat[f], s16])
                            if use_ew:
                                v = v / w16
                            cur = plsc.load_gather(agg, [fsplat[f], d16])
                            plsc.store_scatter(agg, [fsplat[f], d16],
                                               jnp.maximum(v, cur))

                    @pl.when(ndup != 0)
                    def _slow():
                        src_s = _take(s16, perm)
                        w_s = _take(w16, perm) if use_ew else None
                        masks = [(ix, ge & (d_s == _take(d_s, ix)))
                                 for ix, ge in log_steps]
                        run_end = (d_s != _take(d_s, idx_dn)) | lanelast
                        for f in range(fpt):
                            v = plsc.load_gather(h_loc, [fsplat[f], src_s])
                            if use_ew:
                                v = v / w_s
                            for ix, m in masks:
                                v = jnp.maximum(
                                    v, jnp.where(m, _take(v, ix), neg_inf))
                            cur = plsc.load_gather(agg, [fsplat[f], d_s])
                            plsc.store_scatter(agg, [fsplat[f], d_s],
                                               jnp.maximum(v, cur),
                                               mask=run_end)

                nxt = c + 2

                @pl.when(nxt < nchunk)
                def _():
                    fire(slot, nxt)

        @pl.loop(0, NP // L)
        def _(i):
            for f in range(fpt):
                v = agg[f, pl.ds(i * L, L)]
                agg[f, pl.ds(i * L, L)] = jnp.where(v == neg_inf, 0.0, v)

        pltpu.sync_copy(agg, out.at[pl.ds(f0, fpt)])

    return pl.kernel(
        body,
        out_type=jax.ShapeDtypeStruct((H, NP), jnp.float32),
        mesh=mesh,
        scratch_types=scratch,
        compiler_params=pltpu.CompilerParams(needs_layout_passes=False),
    )


def _tc_layer(H, NP, njk, BN):
    grid = (NP // BN,)
    blk = pl.BlockSpec((H, BN), lambda i: (0, i))
    in_specs = [
        blk,
        blk,
        pl.BlockSpec((H, H), lambda i: (0, 0)),
        pl.BlockSpec((H, H), lambda i: (0, 0)),
        pl.BlockSpec((H, 1), lambda i: (0, 0)),
        pl.BlockSpec((H, 1), lambda i: (0, 0)),
    ] + [blk] * njk

    def body(agg_ref, h_ref, w1, w2, b, a, *rest):
        jk, o_ref = rest[:njk], rest[njk]
        z = jnp.dot(w1[...], agg_ref[...], preferred_element_type=jnp.float32)
        z = z + jnp.dot(w2[...], h_ref[...], preferred_element_type=jnp.float32)
        z = z + b[...]
        z = jnp.where(z > 0, z, a[...] * z)
        for r in jk:
            z = jnp.maximum(z, r[...])
        o_ref[...] = z

    return pl.pallas_call(
        body, grid=grid, in_specs=in_specs, out_specs=blk,
        out_shape=jax.ShapeDtypeStruct((H, NP), jnp.float32))


def _tc_head(H, NP, HM, CO, BN):
    grid = (NP // BN,)
    blk_in = pl.BlockSpec((H, BN), lambda i: (0, i))
    blk_out = pl.BlockSpec((CO, BN), lambda i: (0, i))
    in_specs = [
        blk_in,
        pl.BlockSpec((HM, H), lambda i: (0, 0)),
        pl.BlockSpec((HM, 1), lambda i: (0, 0)),
        pl.BlockSpec((HM, 1), lambda i: (0, 0)),
        pl.BlockSpec((CO, HM), lambda i: (0, 0)),
        pl.BlockSpec((CO, 1), lambda i: (0, 0)),
    ]

    def body(h_ref, m1, b1, a1, m2, b2, o_ref):
        z = jnp.dot(m1[...], h_ref[...], preferred_element_type=jnp.float32)
        z = z + b1[...]
        z = jnp.where(z > 0, z, a1[...] * z)
        o_ref[...] = jnp.dot(m2[...], z,
                             preferred_element_type=jnp.float32) + b2[...]

    return pl.pallas_call(
        body, grid=grid, in_specs=in_specs, out_specs=blk_out,
        out_shape=jax.ShapeDtypeStruct((CO, NP), jnp.float32))


def kernel(x, edge_index, edge_attr, batch, params):
    N, D = x.shape
    H = params["g1_Wr"].shape[1]
    E = edge_index.shape[1]

    BN = 2048
    NP = ((max(N + 1, D, H) + BN - 1) // BN) * BN
    EP = ((E + 2 * CHUNK - 1) // (2 * CHUNK)) * (2 * CHUNK)

    src = edge_index[0]
    dst = edge_index[1]
    ea = edge_attr
    if EP != E:
        pad = EP - E
        src = jnp.pad(src, (0, pad))
        dst = jnp.pad(dst, (0, pad), constant_values=N)
        ea = jnp.pad(ea, (0, pad), constant_values=1.0)

    xT = jnp.pad(x.T, ((0, 0), (0, NP - N)))

    segmax_ew = _make_segmax(H, NP, EP, True)
    segmax_pl = _make_segmax(H, NP, EP, False)
    layer = _tc_layer(H, NP, 0, BN)
    layer_jk = _tc_layer(H, NP, 2, BN)
    head = _tc_head(H, NP, 64, 8, BN)

    def col(v, n):
        return jnp.pad(v, (0, n - v.shape[0]))[:, None]

    def gparams(j):
        return (params[f"g{j}_Wr"].T, params[f"g{j}_Wroot"].T,
                col(params[f"g{j}_br"], H), col(params[f"a{j}"], H))

    def sparams(j):
        return (params[f"s{j}_Wl"].T, params[f"s{j}_Wr"].T,
                col(params[f"s{j}_bl"], H), col(params[f"a{j}"], H))

    h1 = layer(segmax_ew(xT, src, dst, ea), xT, *gparams(1))
    h2 = layer(segmax_ew(h1, src, dst, ea), h1, *gparams(2))
    hj = layer_jk(segmax_ew(h2, src, dst, ea), h2, *gparams(3), h1, h2)
    h4 = layer(segmax_pl(hj, src, dst), hj, *sparams(4))
    h5 = layer(segmax_pl(h4, src, dst), h4, *sparams(5))
    hk = layer_jk(segmax_pl(h5, src, dst), h5, *sparams(6), h4, h5)

    hm = 64
    m1 = jnp.pad(params["lin1_W"], ((0, 0), (0, hm - params["lin1_W"].shape[1]))).T
    b1 = col(params["lin1_b"], hm)
    a7 = col(params["a7"], hm)
    m2 = jnp.pad(params["lin2_W"], ((0, hm - params["lin2_W"].shape[0]),
                                    (0, 8 - params["lin2_W"].shape[1]))).T
    b2 = col(params["lin2_b"], 8)

    out = head(hk, m1, b1, a7, m2, b2)
    return out[:3, :N].T


# speedup vs baseline: 2.1791x; 1.8441x over previous
import functools

import jax
import jax.numpy as jnp
from jax import lax
from jax.experimental import pallas as pl
from jax.experimental.pallas import tpu as pltpu
from jax.experimental.pallas import tpu_sc as plsc

L = 16
NCORES = 2
NSUB = 16
NW = NCORES * NSUB
ICH = 2000
OCH = 2000


def _take(x, idx):
    dn = lax.GatherDimensionNumbers(
        offset_dims=(), collapsed_slice_dims=(0,), start_index_map=(0,))
    return lax.gather(x, idx[:, None], dn, (1,),
                      mode=lax.GatherScatterMode.PROMISE_IN_BOUNDS)


def _wid():
    return lax.axis_index("s") * NCORES + lax.axis_index("c")


def _mesh():
    return plsc.VectorSubcoreMesh(core_axis_name="c", subcore_axis_name="s",
                                  num_cores=NCORES, num_subcores=NSUB)


def _make_prep(N, E, TCAP, OVF, SL):
    EPT = E // NW
    assert EPT % ICH == 0 and ICH % L == 0 and OVF == EPT
    nch = EPT // ICH
    STG = SL + OVF

    out_type = (
        jax.ShapeDtypeStruct((NW * SL,), jnp.int32),
        jax.ShapeDtypeStruct((NW * SL,), jnp.int32),
        jax.ShapeDtypeStruct((NW * SL,), jnp.float32),
        jax.ShapeDtypeStruct((NW * OVF,), jnp.int32),
        jax.ShapeDtypeStruct((NW * OVF,), jnp.int32),
        jax.ShapeDtypeStruct((NW * OVF,), jnp.float32),
        jax.ShapeDtypeStruct((NW, L), jnp.int32),
    )
    scratch = [
        pltpu.VMEM((STG,), jnp.int32),
        pltpu.VMEM((STG,), jnp.int32),
        pltpu.VMEM((STG,), jnp.float32),
        pltpu.VMEM((ICH,), jnp.int32),
        pltpu.VMEM((ICH,), jnp.int32),
        pltpu.VMEM((ICH,), jnp.float32),
        pltpu.VMEM((L,), jnp.int32),
        pltpu.VMEM((L,), jnp.int32),
    ]

    def body(src, dst, ea, o_s, o_d, o_w, o_os, o_od, o_ow, o_ofc,
             st_s, st_d, st_w, bs, bd, be, cnt, ofc_r):
        wid = _wid()
        base = wid * EPT

        zeros16 = jnp.zeros((L,), jnp.int32)
        dumdst = jnp.full((L,), N, jnp.int32)
        ones16f = jnp.ones((L,), jnp.float32)

        @pl.loop(0, STG // L)
        def _(i):
            st_s[pl.ds(i * L, L)] = zeros16
            st_d[pl.ds(i * L, L)] = dumdst
            st_w[pl.ds(i * L, L)] = ones16f

        cnt[...] = zeros16

        lane = lax.iota(jnp.int32, L)
        idx_up = jnp.maximum(lane - 1, 0)
        idx_dn = jnp.minimum(lane + 1, L - 1)
        lane0 = lane == 0
        lanelast = lane == (L - 1)

        @pl.loop(0, nch, init_carry=jnp.zeros((L,), jnp.int32))
        def chunkloop(c, ofc):
            pltpu.sync_copy(src.at[pl.ds(base + c * ICH, ICH)], bs)
            pltpu.sync_copy(dst.at[pl.ds(base + c * ICH, ICH)], bd)
            pltpu.sync_copy(ea.at[pl.ds(base + c * ICH, ICH)], be)

            @pl.loop(0, ICH // L, init_carry=ofc)
            def steploop(i, ofc):
                b = i * L
                s16 = bs[pl.ds(b, L)]
                d16 = bd[pl.ds(b, L)]
                w16 = jnp.float32(1.0) / be[pl.ds(b, L)]
                r = jnp.bitwise_and(d16, L - 1)
                r_s, perm = plsc.sort_key_val(r, lane)
                s_s = _take(s16, perm)
                d_s = _take(d16, perm)
                w_s = _take(w16, perm)
                st = (r_s != _take(r_s, idx_up)) | lane0
                rsi = plsc.cummax(jnp.where(st, lane, 0))
                occ = lane - rsi
                cntv = plsc.load_gather(cnt, [r_s])
                rank = cntv + occ
                re = (r_s != _take(r_s, idx_dn)) | lanelast
                plsc.store_scatter(cnt, [r_s], rank + 1, mask=re)
                m_of = rank >= TCAP
                ofr = plsc.cumsum(jnp.where(m_of, 1, 0)) - 1
                pos = jnp.where(m_of, SL + ofc + ofr, rank * L + r_s)
                plsc.store_scatter(st_s, [pos], s_s)
                plsc.store_scatter(st_d, [pos], d_s)
                plsc.store_scatter(st_w, [pos], w_s)
                nof = plsc.all_reduce_population_count(m_of)
                return ofc + nof

            return steploop

        ofc_r[...] = chunkloop
        pltpu.sync_copy(ofc_r, o_ofc.at[wid])
        pltpu.sync_copy(st_s.at[pl.ds(0, SL)], o_s.at[pl.ds(wid * SL, SL)])
        pltpu.sync_copy(st_d.at[pl.ds(0, SL)], o_d.at[pl.ds(wid * SL, SL)])
        pltpu.sync_copy(st_w.at[pl.ds(0, SL)], o_w.at[pl.ds(wid * SL, SL)])
        pltpu.sync_copy(st_s.at[pl.ds(SL, OVF)], o_os.at[pl.ds(wid * OVF, OVF)])
        pltpu.sync_copy(st_d.at[pl.ds(SL, OVF)], o_od.at[pl.ds(wid * OVF, OVF)])
        pltpu.sync_copy(st_w.at[pl.ds(SL, OVF)], o_ow.at[pl.ds(wid * OVF, OVF)])

    return pl.kernel(
        body, out_type=out_type, mesh=_mesh(), scratch_types=scratch,
        compiler_params=pltpu.CompilerParams(needs_layout_passes=False))


def _make_segmax(H, NP, N, TCAP, OVF, SL, use_ew):
    fpt = H // NW
    CH = SL // 4
    assert CH % L == 0 and CH % 8 == 0
    nchunk = NW * 4
    nsteps = CH // L

    scratch = [
        pltpu.VMEM((fpt, NP), jnp.float32),
        pltpu.VMEM((fpt, NP), jnp.float32),
        pltpu.VMEM((CH,), jnp.int32),
        pltpu.VMEM((CH,), jnp.int32),
        pltpu.VMEM((CH,), jnp.int32),
        pltpu.VMEM((CH,), jnp.int32),
        pltpu.VMEM((OCH,), jnp.int32),
        pltpu.VMEM((OCH,), jnp.int32),
        pltpu.VMEM((NW, L), jnp.int32),
        pltpu.SemaphoreType.DMA,
        pltpu.SemaphoreType.DMA,
    ]
    if use_ew:
        scratch += [pltpu.VMEM((CH,), jnp.float32),
                    pltpu.VMEM((CH,), jnp.float32),
                    pltpu.VMEM((OCH,), jnp.float32)]

    def body(hT, s_s, s_d, *rest):
        if use_ew:
            (s_w, v_s, v_d, v_w, ofc, out, h_loc, agg, bs0, bs1, bd0, bd1,
             obs, obd, ofc_v, sem0, sem1, bw0, bw1, obw) = rest
            bwb = (bw0, bw1)
        else:
            (v_s, v_d, ofc, out, h_loc, agg, bs0, bs1, bd0, bd1,
             obs, obd, ofc_v, sem0, sem1) = rest
        bsb = (bs0, bs1)
        bdb = (bd0, bd1)
        sems = (sem0, sem1)

        wid = _wid()
        f0 = wid * fpt

        def fire(slot, c):
            base = c * CH
            pltpu.async_copy(s_s.at[pl.ds(base, CH)], bsb[slot], sems[slot])
            pltpu.async_copy(s_d.at[pl.ds(base, CH)], bdb[slot], sems[slot])
            if use_ew:
                pltpu.async_copy(s_w.at[pl.ds(base, CH)], bwb[slot], sems[slot])

        def drain(slot):
            pltpu.make_async_copy(s_s.at[pl.ds(0, CH)], bsb[slot], sems[slot]).wait()
            pltpu.make_async_copy(s_d.at[pl.ds(0, CH)], bdb[slot], sems[slot]).wait()
            if use_ew:
                pltpu.make_async_copy(s_w.at[pl.ds(0, CH)], bwb[slot], sems[slot]).wait()

        fire(0, 0)
        fire(1, 1)

        pltpu.sync_copy(hT.at[pl.ds(f0, fpt)], h_loc)
        pltpu.sync_copy(ofc, ofc_v)

        neg_inf = jnp.float32(-jnp.inf)
        filler = jnp.full((L,), neg_inf, jnp.float32)

        @pl.loop(0, NP // L)
        def _(i):
            for f in range(fpt):
                agg[f, pl.ds(i * L, L)] = filler

        lane = lax.iota(jnp.int32, L)
        idx_up = jnp.maximum(lane - 1, 0)
        idx_dn = jnp.minimum(lane + 1, L - 1)
        lanepos = lane > 0
        lanelast = lane == (L - 1)
        log_steps = [(jnp.maximum(lane - s, 0), lane >= s) for s in (1, 2, 4, 8)]
        fsplat = [jnp.full((L,), f, jnp.int32) for f in range(fpt)]

        @pl.loop(0, nchunk, step=2)
        def _(c0):
            for slot in range(2):
                c = c0 + slot
                drain(slot)

                @pl.loop(0, nsteps)
                def _(i):
                    b = i * L
                    s16 = bsb[slot][pl.ds(b, L)]
                    d16 = bdb[slot][pl.ds(b, L)]
                    w16 = bwb[slot][pl.ds(b, L)] if use_ew else None
                    for f in range(fpt):
                        v = plsc.load_gather(h_loc, [fsplat[f], s16])
                        if use_ew:
                            v = v * w16
                        cur = plsc.load_gather(agg, [fsplat[f], d16])
                        plsc.store_scatter(agg, [fsplat[f], d16],
                                           jnp.maximum(v, cur))

                nxt = c + 2

                @pl.when(nxt < nchunk)
                def _():
                    fire(slot, nxt)

        def dup_step(sbuf, dbuf, wbuf, i):
            b = i * L
            s16 = sbuf[pl.ds(b, L)]
            d16 = dbuf[pl.ds(b, L)]
            w16 = wbuf[pl.ds(b, L)] if use_ew else None
            d_sv, perm = plsc.sort_key_val(d16, lane)
            src_s = _take(s16, perm)
            w_s = _take(w16, perm) if use_ew else None
            masks = [(ix, ge & (d_sv == _take(d_sv, ix)))
                     for ix, ge in log_steps]
            run_end = (d_sv != _take(d_sv, idx_dn)) | lanelast
            for f in range(fpt):
                v = plsc.load_gather(h_loc, [fsplat[f], src_s])
                if use_ew:
                    v = v * w_s
                for ix, m in masks:
                    v = jnp.maximum(v, jnp.where(m, _take(v, ix), neg_inf))
                cur = plsc.load_gather(agg, [fsplat[f], d_sv])
                plsc.store_scatter(agg, [fsplat[f], d_sv],
                                   jnp.maximum(v, cur), mask=run_end)

        @pl.loop(0, NW)
        def _(t):
            n_t = ofc_v[t, pl.ds(0, L)][0]

            @pl.when(n_t > 0)
            def _():
                nst = (n_t + (L - 1)) // L
                nchv = (n_t + (OCH - 1)) // OCH

                @pl.loop(0, nchv)
                def _(c):
                    pltpu.sync_copy(v_s.at[pl.ds(t * OVF + c * OCH, OCH)], obs)
                    pltpu.sync_copy(v_d.at[pl.ds(t * OVF + c * OCH, OCH)], obd)
                    if use_ew:
                        pltpu.sync_copy(v_w.at[pl.ds(t * OVF + c * OCH, OCH)], obw)
                    ns = jnp.minimum(OCH // L, nst - c * (OCH // L))

                    @pl.loop(0, ns)
                    def _(i):
                        dup_step(obs, obd, obw if use_ew else None, i)

        @pl.loop(0, NP // L)
        def _(i):
            for f in range(fpt):
                v = agg[f, pl.ds(i * L, L)]
                agg[f, pl.ds(i * L, L)] = jnp.where(v == neg_inf, 0.0, v)

        pltpu.sync_copy(agg, out.at[pl.ds(f0, fpt)])

    return pl.kernel(
        body,
        out_type=jax.ShapeDtypeStruct((H, NP), jnp.float32),
        mesh=_mesh(),
        scratch_types=scratch,
        compiler_params=pltpu.CompilerParams(needs_layout_passes=False))


def _tc_layer(H, NP, njk, BN):
    grid = (NP // BN,)
    blk = pl.BlockSpec((H, BN), lambda i: (0, i))
    in_specs = [
        blk,
        blk,
        pl.BlockSpec((H, H), lambda i: (0, 0)),
        pl.BlockSpec((H, H), lambda i: (0, 0)),
        pl.BlockSpec((H, 1), lambda i: (0, 0)),
        pl.BlockSpec((H, 1), lambda i: (0, 0)),
    ] + [blk] * njk

    def body(agg_ref, h_ref, w1, w2, b, a, *rest):
        jk, o_ref = rest[:njk], rest[njk]
        z = jnp.dot(w1[...], agg_ref[...], preferred_element_type=jnp.float32)
        z = z + jnp.dot(w2[...], h_ref[...], preferred_element_type=jnp.float32)
        z = z + b[...]
        z = jnp.where(z > 0, z, a[...] * z)
        for r in jk:
            z = jnp.maximum(z, r[...])
        o_ref[...] = z

    return pl.pallas_call(
        body, grid=grid, in_specs=in_specs, out_specs=blk,
        out_shape=jax.ShapeDtypeStruct((H, NP), jnp.float32))


def _tc_head(H, NP, HM, CO, BN):
    grid = (NP // BN,)
    blk_in = pl.BlockSpec((H, BN), lambda i: (0, i))
    blk_out = pl.BlockSpec((CO, BN), lambda i: (0, i))
    in_specs = [
        blk_in,
        pl.BlockSpec((HM, H), lambda i: (0, 0)),
        pl.BlockSpec((HM, 1), lambda i: (0, 0)),
        pl.BlockSpec((HM, 1), lambda i: (0, 0)),
        pl.BlockSpec((CO, HM), lambda i: (0, 0)),
        pl.BlockSpec((CO, 1), lambda i: (0, 0)),
    ]

    def body(h_ref, m1, b1, a1, m2, b2, o_ref):
        z = jnp.dot(m1[...], h_ref[...], preferred_element_type=jnp.float32)
        z = z + b1[...]
        z = jnp.where(z > 0, z, a1[...] * z)
        o_ref[...] = jnp.dot(m2[...], z,
                             preferred_element_type=jnp.float32) + b2[...]

    return pl.pallas_call(
        body, grid=grid, in_specs=in_specs, out_specs=blk_out,
        out_shape=jax.ShapeDtypeStruct((CO, NP), jnp.float32))


def kernel(x, edge_index, edge_attr, batch, params):
    N, D = x.shape
    H = params["g1_Wr"].shape[1]
    E = edge_index.shape[1]

    BN = 2048
    NP = ((max(N + 1, D, H) + BN - 1) // BN) * BN

    EBLK = NW * ICH
    EP = ((E + EBLK - 1) // EBLK) * EBLK
    EPT = EP // NW
    TCAP = ((EPT // L) * 27 // 25 + 3) // 4 * 4
    SL = L * TCAP
    OVF = EPT

    src = edge_index[0]
    dst = edge_index[1]
    ea = edge_attr
    if EP != E:
        pad = EP - E
        src = jnp.pad(src, (0, pad))
        dst = jnp.pad(dst, (0, pad), constant_values=N)
        ea = jnp.pad(ea, (0, pad), constant_values=1.0)

    xT = jnp.pad(x.T, ((0, 0), (0, NP - N)))

    prep = _make_prep(N, EP, TCAP, OVF, SL)
    st_s, st_d, st_w, ov_s, ov_d, ov_w, ofc = prep(src, dst, ea)

    segmax_ew = _make_segmax(H, NP, N, TCAP, OVF, SL, True)
    segmax_pl = _make_segmax(H, NP, N, TCAP, OVF, SL, False)
    gagg = lambda h: segmax_ew(h, st_s, st_d, st_w, ov_s, ov_d, ov_w, ofc)
    sagg = lambda h: segmax_pl(h, st_s, st_d, ov_s, ov_d, ofc)

    layer = _tc_layer(H, NP, 0, BN)
    layer_jk = _tc_layer(H, NP, 2, BN)
    head = _tc_head(H, NP, 64, 8, BN)

    def col(v, n):
        return jnp.pad(v, (0, n - v.shape[0]))[:, None]

    def gparams(j):
        return (params[f"g{j}_Wr"].T, params[f"g{j}_Wroot"].T,
                col(params[f"g{j}_br"], H), col(params[f"a{j}"], H))

    def sparams(j):
        return (params[f"s{j}_Wl"].T, params[f"s{j}_Wr"].T,
                col(params[f"s{j}_bl"], H), col(params[f"a{j}"], H))

    h1 = layer(gagg(xT), xT, *gparams(1))
    h2 = layer(gagg(h1), h1, *gparams(2))
    hj = layer_jk(gagg(h2), h2, *gparams(3), h1, h2)
    h4 = layer(sagg(hj), hj, *sparams(4))
    h5 = layer(sagg(h4), h4, *sparams(5))
    hk = layer_jk(sagg(h5), h5, *sparams(6), h4, h5)

    hm = 64
    m1 = jnp.pad(params["lin1_W"], ((0, 0), (0, hm - params["lin1_W"].shape[1]))).T
    b1 = col(params["lin1_b"], hm)
    a7 = col(params["a7"], hm)
    m2 = jnp.pad(params["lin2_W"], ((0, hm - params["lin2_W"].shape[0]),
                                    (0, 8 - params["lin2_W"].shape[1]))).T
    b2 = col(params["lin2_b"], 8)

    out = head(hk, m1, b1, a7, m2, b2)
    return out[:3, :N].T


# speedup vs baseline: 3.4712x; 1.5930x over previous
import functools

import jax
import jax.numpy as jnp
from jax import lax
from jax.experimental import pallas as pl
from jax.experimental.pallas import tpu as pltpu
from jax.experimental.pallas import tpu_sc as plsc

L = 16
NCORES = 2
NSUB = 16
NW = NCORES * NSUB
ICH = 2000
OCH = 2000


def _take(x, idx):
    dn = lax.GatherDimensionNumbers(
        offset_dims=(), collapsed_slice_dims=(0,), start_index_map=(0,))
    return lax.gather(x, idx[:, None], dn, (1,),
                      mode=lax.GatherScatterMode.PROMISE_IN_BOUNDS)


def _wid():
    return lax.axis_index("s") * NCORES + lax.axis_index("c")


def _mesh():
    return plsc.VectorSubcoreMesh(core_axis_name="c", subcore_axis_name="s",
                                  num_cores=NCORES, num_subcores=NSUB)


def _make_prep(N, E, TCAP, OVF, SL):
    EPT = E // NW
    assert EPT % ICH == 0 and ICH % L == 0 and OVF == EPT
    nch = EPT // ICH
    STG = SL + OVF

    out_type = (
        jax.ShapeDtypeStruct((NW * SL,), jnp.int32),
        jax.ShapeDtypeStruct((NW * SL,), jnp.int32),
        jax.ShapeDtypeStruct((NW * SL,), jnp.float32),
        jax.ShapeDtypeStruct((NW * OVF,), jnp.int32),
        jax.ShapeDtypeStruct((NW * OVF,), jnp.int32),
        jax.ShapeDtypeStruct((NW * OVF,), jnp.float32),
        jax.ShapeDtypeStruct((NW, L), jnp.int32),
    )
    scratch = [
        pltpu.VMEM((STG,), jnp.int32),
        pltpu.VMEM((STG,), jnp.int32),
        pltpu.VMEM((STG,), jnp.float32),
        pltpu.VMEM((ICH,), jnp.int32),
        pltpu.VMEM((ICH,), jnp.int32),
        pltpu.VMEM((ICH,), jnp.float32),
        pltpu.VMEM((L,), jnp.int32),
        pltpu.VMEM((L,), jnp.int32),
    ]

    def body(src, dst, ea, o_s, o_d, o_w, o_os, o_od, o_ow, o_ofc,
             st_s, st_d, st_w, bs, bd, be, cnt, ofc_r):
        wid = _wid()
        base = wid * EPT

        zeros16 = jnp.zeros((L,), jnp.int32)
        dumdst = jnp.full((L,), N, jnp.int32)
        ones16f = jnp.ones((L,), jnp.float32)

        @pl.loop(0, STG // L)
        def _(i):
            st_s[pl.ds(i * L, L)] = zeros16
            st_d[pl.ds(i * L, L)] = dumdst
            st_w[pl.ds(i * L, L)] = ones16f

        cnt[...] = zeros16

        lane = lax.iota(jnp.int32, L)
        idx_up = jnp.maximum(lane - 1, 0)
        idx_dn = jnp.minimum(lane + 1, L - 1)
        lane0 = lane == 0
        lanelast = lane == (L - 1)

        @pl.loop(0, nch, init_carry=jnp.zeros((L,), jnp.int32))
        def chunkloop(c, ofc):
            pltpu.sync_copy(src.at[pl.ds(base + c * ICH, ICH)], bs)
            pltpu.sync_copy(dst.at[pl.ds(base + c * ICH, ICH)], bd)
            pltpu.sync_copy(ea.at[pl.ds(base + c * ICH, ICH)], be)

            @pl.loop(0, ICH // L, init_carry=ofc)
            def steploop(i, ofc):
                b = i * L
                s16 = bs[pl.ds(b, L)]
                d16 = bd[pl.ds(b, L)]
                w16 = jnp.float32(1.0) / be[pl.ds(b, L)]
                r = jnp.bitwise_and(d16, L - 1)
                r_s, perm = plsc.sort_key_val(r, lane)
                s_s = _take(s16, perm)
                d_s = _take(d16, perm)
                w_s = _take(w16, perm)
                st = (r_s != _take(r_s, idx_up)) | lane0
                rsi = plsc.cummax(jnp.where(st, lane, 0))
                occ = lane - rsi
                cntv = plsc.load_gather(cnt, [r_s])
                rank = cntv + occ
                re = (r_s != _take(r_s, idx_dn)) | lanelast
                plsc.store_scatter(cnt, [r_s], rank + 1, mask=re)
                m_of = rank >= TCAP
                ofr = plsc.cumsum(jnp.where(m_of, 1, 0)) - 1
                pos = jnp.where(m_of, SL + ofc + ofr, rank * L + r_s)
                plsc.store_scatter(st_s, [pos], s_s)
                plsc.store_scatter(st_d, [pos], d_s)
                plsc.store_scatter(st_w, [pos], w_s)
                nof = plsc.all_reduce_population_count(m_of)
                return ofc + nof

            return steploop

        ofc_r[...] = chunkloop
        pltpu.sync_copy(ofc_r, o_ofc.at[wid])
        pltpu.sync_copy(st_s.at[pl.ds(0, SL)], o_s.at[pl.ds(wid * SL, SL)])
        pltpu.sync_copy(st_d.at[pl.ds(0, SL)], o_d.at[pl.ds(wid * SL, SL)])
        pltpu.sync_copy(st_w.at[pl.ds(0, SL)], o_w.at[pl.ds(wid * SL, SL)])
        pltpu.sync_copy(st_s.at[pl.ds(SL, OVF)], o_os.at[pl.ds(wid * OVF, OVF)])
        pltpu.sync_copy(st_d.at[pl.ds(SL, OVF)], o_od.at[pl.ds(wid * OVF, OVF)])
        pltpu.sync_copy(st_w.at[pl.ds(SL, OVF)], o_ow.at[pl.ds(wid * OVF, OVF)])

    return pl.kernel(
        body, out_type=out_type, mesh=_mesh(), scratch_types=scratch,
        compiler_params=pltpu.CompilerParams(needs_layout_passes=False))


def _make_segmax(H, NP, N, TCAP, OVF, SL, use_ew):
    fpt = H // NW
    CH = SL // 4
    assert CH % L == 0 and CH % 8 == 0
    nchunk = NW * 4
    nsteps = CH // L

    scratch = [
        pltpu.VMEM((fpt, NP), jnp.float32),
        pltpu.VMEM((fpt, NP), jnp.float32),
        pltpu.VMEM((CH,), jnp.int32),
        pltpu.VMEM((CH,), jnp.int32),
        pltpu.VMEM((CH,), jnp.int32),
        pltpu.VMEM((CH,), jnp.int32),
        pltpu.VMEM((OCH,), jnp.int32),
        pltpu.VMEM((OCH,), jnp.int32),
        pltpu.VMEM((NW, L), jnp.int32),
        pltpu.SemaphoreType.DMA,
        pltpu.SemaphoreType.DMA,
    ]
    if use_ew:
        scratch += [pltpu.VMEM((CH,), jnp.float32),
                    pltpu.VMEM((CH,), jnp.float32),
                    pltpu.VMEM((OCH,), jnp.float32)]

    def body(hT, s_s, s_d, *rest):
        if use_ew:
            (s_w, v_s, v_d, v_w, ofc, out, h_loc, agg, bs0, bs1, bd0, bd1,
             obs, obd, ofc_v, sem0, sem1, bw0, bw1, obw) = rest
            bwb = (bw0, bw1)
        else:
            (v_s, v_d, ofc, out, h_loc, agg, bs0, bs1, bd0, bd1,
             obs, obd, ofc_v, sem0, sem1) = rest
        bsb = (bs0, bs1)
        bdb = (bd0, bd1)
        sems = (sem0, sem1)

        wid = _wid()
        f0 = wid * fpt

        def fire(slot, c):
            base = c * CH
            pltpu.async_copy(s_s.at[pl.ds(base, CH)], bsb[slot], sems[slot])
            pltpu.async_copy(s_d.at[pl.ds(base, CH)], bdb[slot], sems[slot])
            if use_ew:
                pltpu.async_copy(s_w.at[pl.ds(base, CH)], bwb[slot], sems[slot])

        def drain(slot):
            pltpu.make_async_copy(s_s.at[pl.ds(0, CH)], bsb[slot], sems[slot]).wait()
            pltpu.make_async_copy(s_d.at[pl.ds(0, CH)], bdb[slot], sems[slot]).wait()
            if use_ew:
                pltpu.make_async_copy(s_w.at[pl.ds(0, CH)], bwb[slot], sems[slot]).wait()

        fire(0, 0)
        fire(1, 1)

        pltpu.sync_copy(hT.at[pl.ds(f0, fpt)], h_loc)
        pltpu.sync_copy(ofc, ofc_v)

        neg_inf = jnp.float32(-jnp.inf)
        filler = jnp.full((L,), neg_inf, jnp.float32)

        @pl.loop(0, NP // L)
        def _(i):
            for f in range(fpt):
                agg[f, pl.ds(i * L, L)] = filler

        lane = lax.iota(jnp.int32, L)
        idx_up = jnp.maximum(lane - 1, 0)
        idx_dn = jnp.minimum(lane + 1, L - 1)
        lanepos = lane > 0
        lanelast = lane == (L - 1)
        log_steps = [(jnp.maximum(lane - s, 0), lane >= s) for s in (1, 2, 4, 8)]
        fsplat = [jnp.full((L,), f, jnp.int32) for f in range(fpt)]

        @pl.loop(0, nchunk, step=2)
        def _(c0):
            for slot in range(2):
                c = c0 + slot
                drain(slot)

                @pl.loop(0, nsteps, unroll=2)
                def _(i):
                    b = i * L
                    s16 = bsb[slot][pl.ds(b, L)]
                    d16 = bdb[slot][pl.ds(b, L)]
                    w16 = bwb[slot][pl.ds(b, L)] if use_ew else None
                    vs = [plsc.load_gather(h_loc, [fsplat[f], s16])
                          for f in range(fpt)]
                    if use_ew:
                        vs = [v * w16 for v in vs]
                    curs = [plsc.load_gather(agg, [fsplat[f], d16])
                            for f in range(fpt)]
                    for f in range(fpt):
                        plsc.store_scatter(agg, [fsplat[f], d16],
                                           jnp.maximum(vs[f], curs[f]))

                nxt = c + 2

                @pl.when(nxt < nchunk)
                def _():
                    fire(slot, nxt)

        def dup_step(sbuf, dbuf, wbuf, i):
            b = i * L
            s16 = sbuf[pl.ds(b, L)]
            d16 = dbuf[pl.ds(b, L)]
            w16 = wbuf[pl.ds(b, L)] if use_ew else None
            d_sv, perm = plsc.sort_key_val(d16, lane)
            src_s = _take(s16, perm)
            w_s = _take(w16, perm) if use_ew else None
            masks = [(ix, ge & (d_sv == _take(d_sv, ix)))
                     for ix, ge in log_steps]
            run_end = (d_sv != _take(d_sv, idx_dn)) | lanelast
            for f in range(fpt):
                v = plsc.load_gather(h_loc, [fsplat[f], src_s])
                if use_ew:
                    v = v * w_s
                for ix, m in masks:
                    v = jnp.maximum(v, jnp.where(m, _take(v, ix), neg_inf))
                cur = plsc.load_gather(agg, [fsplat[f], d_sv])
                plsc.store_scatter(agg, [fsplat[f], d_sv],
                                   jnp.maximum(v, cur), mask=run_end)

        @pl.loop(0, NW)
        def _(t):
            n_t = ofc_v[t, pl.ds(0, L)][0]

            @pl.when(n_t > 0)
            def _():
                nst = (n_t + (L - 1)) // L
                nchv = (n_t + (OCH - 1)) // OCH

                @pl.loop(0, nchv)
                def _(c):
                    pltpu.sync_copy(v_s.at[pl.ds(t * OVF + c * OCH, OCH)], obs)
                    pltpu.sync_copy(v_d.at[pl.ds(t * OVF + c * OCH, OCH)], obd)
                    if use_ew:
                        pltpu.sync_copy(v_w.at[pl.ds(t * OVF + c * OCH, OCH)], obw)
                    ns = jnp.minimum(OCH // L, nst - c * (OCH // L))

                    @pl.loop(0, ns)
                    def _(i):
                        dup_step(obs, obd, obw if use_ew else None, i)

        @pl.loop(0, NP // L)
        def _(i):
            for f in range(fpt):
                v = agg[f, pl.ds(i * L, L)]
                agg[f, pl.ds(i * L, L)] = jnp.where(v == neg_inf, 0.0, v)

        pltpu.sync_copy(agg, out.at[pl.ds(f0, fpt)])

    return pl.kernel(
        body,
        out_type=jax.ShapeDtypeStruct((H, NP), jnp.float32),
        mesh=_mesh(),
        scratch_types=scratch,
        compiler_params=pltpu.CompilerParams(needs_layout_passes=False))


def _tc_layer(H, NP, njk, BN):
    grid = (NP // BN,)
    blk = pl.BlockSpec((H, BN), lambda i: (0, i))
    in_specs = [
        blk,
        blk,
        pl.BlockSpec((H, H), lambda i: (0, 0)),
        pl.BlockSpec((H, H), lambda i: (0, 0)),
        pl.BlockSpec((H, 1), lambda i: (0, 0)),
        pl.BlockSpec((H, 1), lambda i: (0, 0)),
    ] + [blk] * njk

    def body(agg_ref, h_ref, w1, w2, b, a, *rest):
        jk, o_ref = rest[:njk], rest[njk]
        z = jnp.dot(w1[...], agg_ref[...], preferred_element_type=jnp.float32)
        z = z + jnp.dot(w2[...], h_ref[...], preferred_element_type=jnp.float32)
        z = z + b[...]
        z = jnp.where(z > 0, z, a[...] * z)
        for r in jk:
            z = jnp.maximum(z, r[...])
        o_ref[...] = z

    return pl.pallas_call(
        body, grid=grid, in_specs=in_specs, out_specs=blk,
        out_shape=jax.ShapeDtypeStruct((H, NP), jnp.float32))


def _tc_head(H, NP, HM, CO, BN):
    grid = (NP // BN,)
    blk_in = pl.BlockSpec((H, BN), lambda i: (0, i))
    blk_out = pl.BlockSpec((CO, BN), lambda i: (0, i))
    in_specs = [
        blk_in,
        pl.BlockSpec((HM, H), lambda i: (0, 0)),
        pl.BlockSpec((HM, 1), lambda i: (0, 0)),
        pl.BlockSpec((HM, 1), lambda i: (0, 0)),
        pl.BlockSpec((CO, HM), lambda i: (0, 0)),
        pl.BlockSpec((CO, 1), lambda i: (0, 0)),
    ]

    def body(h_ref, m1, b1, a1, m2, b2, o_ref):
        z = jnp.dot(m1[...], h_ref[...], preferred_element_type=jnp.float32)
        z = z + b1[...]
        z = jnp.where(z > 0, z, a1[...] * z)
        o_ref[...] = jnp.dot(m2[...], z,
                             preferred_element_type=jnp.float32) + b2[...]

    return pl.pallas_call(
        body, grid=grid, in_specs=in_specs, out_specs=blk_out,
        out_shape=jax.ShapeDtypeStruct((CO, NP), jnp.float32))


def kernel(x, edge_index, edge_attr, batch, params):
    N, D = x.shape
    H = params["g1_Wr"].shape[1]
    E = edge_index.shape[1]

    BN = 2048
    NP = ((max(N + 1, D, H) + BN - 1) // BN) * BN

    EBLK = NW * ICH
    EP = ((E + EBLK - 1) // EBLK) * EBLK
    EPT = EP // NW
    TCAP = ((EPT // L) * 27 // 25 + 3) // 4 * 4
    SL = L * TCAP
    OVF = EPT

    src = edge_index[0]
    dst = edge_index[1]
    ea = edge_attr
    if EP != E:
        pad = EP - E
        src = jnp.pad(src, (0, pad))
        dst = jnp.pad(dst, (0, pad), constant_values=N)
        ea = jnp.pad(ea, (0, pad), constant_values=1.0)

    xT = jnp.pad(x.T, ((0, 0), (0, NP - N)))

    prep = _make_prep(N, EP, TCAP, OVF, SL)
    st_s, st_d, st_w, ov_s, ov_d, ov_w, ofc = prep(src, dst, ea)

    segmax_ew = _make_segmax(H, NP, N, TCAP, OVF, SL, True)
    segmax_pl = _make_segmax(H, NP, N, TCAP, OVF, SL, False)
    gagg = lambda h: segmax_ew(h, st_s, st_d, st_w, ov_s, ov_d, ov_w, ofc)
    sagg = lambda h: segmax_pl(h, st_s, st_d, ov_s, ov_d, ofc)

    layer = _tc_layer(H, NP, 0, BN)
    layer_jk = _tc_layer(H, NP, 2, BN)
    head = _tc_head(H, NP, 64, 8, BN)

    def col(v, n):
        return jnp.pad(v, (0, n - v.shape[0]))[:, None]

    def gparams(j):
        return (params[f"g{j}_Wr"].T, params[f"g{j}_Wroot"].T,
                col(params[f"g{j}_br"], H), col(params[f"a{j}"], H))

    def sparams(j):
        return (params[f"s{j}_Wl"].T, params[f"s{j}_Wr"].T,
                col(params[f"s{j}_bl"], H), col(params[f"a{j}"], H))

    h1 = layer(gagg(xT), xT, *gparams(1))
    h2 = layer(gagg(h1), h1, *gparams(2))
    hj = layer_jk(gagg(h2), h2, *gparams(3), h1, h2)
    h4 = layer(sagg(hj), hj, *sparams(4))
    h5 = layer(sagg(h4), h4, *sparams(5))
    hk = layer_jk(sagg(h5), h5, *sparams(6), h4, h5)

    hm = 64
    m1 = jnp.pad(params["lin1_W"], ((0, 0), (0, hm - params["lin1_W"].shape[1]))).T
    b1 = col(params["lin1_b"], hm)
    a7 = col(params["a7"], hm)
    m2 = jnp.pad(params["lin2_W"], ((0, hm - params["lin2_W"].shape[0]),
                                    (0, 8 - params["lin2_W"].shape[1]))).T
    b2 = col(params["lin2_b"], 8)

    out = head(hk, m1, b1, a7, m2, b2)
    return out[:3, :N].T
